# Initial kernel scaffold; baseline (speedup 1.0000x reference)
import jax
import jax.numpy as jnp
from jax.experimental import pallas as pl


def kernel(x, conv1_w, conv1_b, conv2_w, conv2_b, fc1_w, fc1_b, fc2_w, fc2_b):
    raise NotImplementedError("write your pallas kernel here")


# speedup vs baseline: 520.1266x; 520.1266x over previous
import jax
import jax.numpy as jnp
from jax.experimental import pallas as pl
from jax.experimental.pallas import tpu as pltpu

_BB = 128
_VMEM_LIMIT = 100 * 1024 * 1024


def _fused_kernel(x_ref, w1_ref, b1_ref, w2_ref, b2_ref,
                  fc1_ref, fb1_ref, fc2_ref, fb2_ref, o_ref,
                  s1_ref, s2_ref):
    bb = x_ref.shape[0]

    x = x_ref[...]
    xa = jnp.concatenate([x[:, kh:kh + 24, :] for kh in range(5)], axis=-1)
    xa = xa.reshape(bb * 24, 140)
    y1 = jnp.dot(xa, w1_ref[...], preferred_element_type=jnp.float32)
    y1 = jnp.maximum(y1 + b1_ref[...], 0.0)
    y1 = y1.reshape(bb, 24, 768)

    w1max = jnp.maximum(y1[:, :, :384], y1[:, :, 384:])
    s1_ref[...] = w1max.reshape(bb, 24, 3, 128)
    p1 = jnp.maximum(s1_ref[:, pl.ds(0, 12, stride=2), :, :],
                     s1_ref[:, pl.ds(1, 12, stride=2), :, :])
    p1 = p1.reshape(bb, 12, 384)

    xb = jnp.concatenate([p1[:, kh:kh + 8, :] for kh in range(5)], axis=-1)
    xb = xb.reshape(bb * 8, 1920)
    y2 = jnp.dot(xb, w2_ref[...], preferred_element_type=jnp.float32)
    y2 = jnp.maximum(y2 + b2_ref[...], 0.0)
    y2 = y2.reshape(bb, 8, 512)

    w2max = jnp.maximum(y2[:, :, :256], y2[:, :, 256:])
    s2_ref[...] = w2max.reshape(bb, 8, 2, 128)
    p2 = jnp.maximum(s2_ref[:, pl.ds(0, 4, stride=2), :, :],
                     s2_ref[:, pl.ds(1, 4, stride=2), :, :])
    p2 = p2.reshape(bb, 4, 256)

    hh = jnp.dot(p2[:, 0, :], fc1_ref[0], preferred_element_type=jnp.float32)
    for ph in range(1, 4):
        hh = hh + jnp.dot(p2[:, ph, :], fc1_ref[ph],
                          preferred_element_type=jnp.float32)
    hh = jnp.maximum(hh + fb1_ref[...], 0.0)

    logits = jnp.dot(hh, fc2_ref[...], preferred_element_type=jnp.float32)
    logits = logits + fb2_ref[...]
    m = jnp.max(logits, axis=-1, keepdims=True)
    e = jnp.exp(logits - m)
    o_ref[...] = (e / jnp.sum(e, axis=-1, keepdims=True)).astype(o_ref.dtype)


def _band_weights(conv1_w, conv2_w):
    w1r = conv1_w.reshape(5, 5, 32)
    d1 = jnp.arange(28)[:, None] - jnp.arange(24)[None, :]
    v1 = (d1 >= 0) & (d1 < 5)
    W1 = w1r[:, jnp.clip(d1, 0, 4), :]
    W1 = jnp.where(v1[None, :, :, None], W1, 0.0)
    W1 = W1.reshape(5, 28, 12, 2, 32)
    W1 = W1.transpose(0, 1, 3, 4, 2).reshape(140, 768)

    w2r = conv2_w.reshape(32, 5, 5, 64).transpose(1, 0, 2, 3)
    d2 = jnp.arange(12)[:, None] - jnp.arange(8)[None, :]
    v2 = (d2 >= 0) & (d2 < 5)
    W2 = w2r[:, :, jnp.clip(d2, 0, 4), :]
    W2 = jnp.where(v2[None, None, :, :, None], W2, 0.0)
    W2 = W2.reshape(5, 32, 12, 4, 2, 64)
    W2 = W2.transpose(0, 1, 2, 4, 5, 3).reshape(1920, 512)
    return W1, W2


def kernel(x, conv1_w, conv1_b, conv2_w, conv2_b, fc1_w, fc1_b, fc2_w, fc2_b):
    n = x.shape[0]
    xr = x.reshape(n, 28, 28)
    W1, W2 = _band_weights(conv1_w, conv2_w)
    b1 = jnp.tile(jnp.repeat(conv1_b[0], 12), 2).reshape(1, 768)
    b2 = jnp.tile(jnp.repeat(conv2_b[0], 4), 2).reshape(1, 512)
    fc1p = fc1_w.reshape(4, 4, 64, 128).transpose(0, 2, 1, 3).reshape(4, 256, 128)

    bb = _BB if n % _BB == 0 else n
    grid = (n // bb,)
    return pl.pallas_call(
        _fused_kernel,
        out_shape=jax.ShapeDtypeStruct((n, 10), x.dtype),
        grid=grid,
        in_specs=[
            pl.BlockSpec((bb, 28, 28), lambda i: (i, 0, 0)),
            pl.BlockSpec((140, 768), lambda i: (0, 0)),
            pl.BlockSpec((1, 768), lambda i: (0, 0)),
            pl.BlockSpec((1920, 512), lambda i: (0, 0)),
            pl.BlockSpec((1, 512), lambda i: (0, 0)),
            pl.BlockSpec((4, 256, 128), lambda i: (0, 0, 0)),
            pl.BlockSpec((1, 128), lambda i: (0, 0)),
            pl.BlockSpec((128, 10), lambda i: (0, 0)),
            pl.BlockSpec((1, 10), lambda i: (0, 0)),
        ],
        out_specs=pl.BlockSpec((bb, 10), lambda i: (i, 0)),
        scratch_shapes=[
            pltpu.VMEM((bb, 24, 3, 128), jnp.float32),
            pltpu.VMEM((bb, 8, 2, 128), jnp.float32),
        ],
        compiler_params=pltpu.CompilerParams(
            dimension_semantics=("parallel",),
            vmem_limit_bytes=_VMEM_LIMIT,
        ),
        cost_estimate=pl.CostEstimate(
            flops=2 * n * (24 * 140 * 768 + 8 * 1920 * 512 + 1024 * 128 + 128 * 10),
            transcendentals=n * 10,
            bytes_accessed=4 * (n * 28 * 28 + n * 10),
        ),
    )(xr, W1, b1, W2, b2, fc1p, fc1_b, fc2_w, fc2_b)
